# baseline (speedup 1.0000x reference)
import functools

import jax
import jax.numpy as jnp
from jax import lax
from jax.experimental import pallas as pl
from jax.experimental.pallas import tpu as pltpu
from jax.experimental.pallas import tpu_sc as plsc

N = 10000
K = 32
H = 128
HH = H // 2
TCOLS = 128
E = N * K

BN = 200
EB = BN * K
NBLK = N // BN

NCH = 2
E2 = E // NCH
N2 = N // NCH
NBLK2 = NBLK // NCH

NW = 32
PER_TILE = E2 // NW
CHUNK = 40
DEPTH = 5
NCHUNKS = PER_TILE // CHUNK
PAIRS = (NCHUNKS - DEPTH) // (2 * DEPTH)


def _dotbf(a, b):
    return jnp.dot(a.astype(jnp.bfloat16), b.astype(jnp.bfloat16),
                   preferred_element_type=jnp.float32)


def _silu(v):
    hv = 0.5 * v
    return hv * jnp.tanh(hv) + hv


def _silu_bf(v):
    hv = jnp.bfloat16(0.5) * v.astype(jnp.bfloat16)
    return hv * jnp.tanh(hv) + hv


def _prep_body(h_ref, x_ref, we1_ref, be1_ref, s_ref, t_ref):
    h = h_ref[0]
    x = x_ref[0]
    xsq = jnp.sum(x * x, axis=1, keepdims=True)
    wd = we1_ref[2 * H:2 * H + 1]
    xsqwd = xsq * wd
    s_ref[0] = (
        jnp.dot(h, we1_ref[:H], preferred_element_type=jnp.float32)
        + be1_ref[...] + xsqwd
    )
    bn = _dotbf(h, we1_ref[H:2 * H]) + xsqwd
    bnb = lax.bitcast_convert_type(bn, jnp.int32)
    lo16 = lax.shift_right_logical(bnb[:, :HH] + 0x8000, 16)
    hi16 = (bnb[:, HH:] + 0x8000) & (-65536)
    t_ref[:, :HH] = lax.bitcast_convert_type(hi16 | lo16, jnp.float32)
    t_ref[:, HH:HH + 3] = x
    t_ref[:, HH + 3:HH + 4] = jnp.ones((N, 1), jnp.float32)
    t_ref[:, HH + 4:] = jnp.zeros((N, TCOLS - HH - 4), jnp.float32)


@functools.cache
def _make_gather(off):
    mesh = plsc.VectorSubcoreMesh(core_axis_name="c", subcore_axis_name="s")
    scratch = [pltpu.VMEM((PER_TILE,), jnp.int32)]
    scratch += [pltpu.VMEM((CHUNK, TCOLS), jnp.float32)
                for _ in range(2 * DEPTH)]
    scratch += [pltpu.SemaphoreType.DMA for _ in range(4 * DEPTH)]

    @functools.partial(
        pl.kernel,
        mesh=mesh,
        out_type=jax.ShapeDtypeStruct((E2, TCOLS), jnp.float32),
        scratch_types=scratch,
    )
    def gather_k(t_hbm, eidx_hbm, g_hbm, idx_v, *rest):
        bufs = rest[:2 * DEPTH]
        gsems = rest[2 * DEPTH:4 * DEPTH]
        ssems = rest[4 * DEPTH:]
        wid = lax.axis_index("s") * 2 + lax.axis_index("c")
        base = wid * PER_TILE
        pltpu.sync_copy(eidx_hbm.at[pl.ds(off + base, PER_TILE)], idx_v)

        def gstart(c, b):
            pltpu.async_copy(
                t_hbm.at[idx_v.at[pl.ds(c * CHUNK, CHUNK)]], bufs[b],
                gsems[b])

        def gwait(c, b):
            pltpu.make_async_copy(
                t_hbm.at[idx_v.at[pl.ds(c * CHUNK, CHUNK)]], bufs[b],
                gsems[b]).wait()

        def sstart(c, b):
            pltpu.async_copy(
                bufs[b], g_hbm.at[pl.ds(base + c * CHUNK, CHUNK)], ssems[b])

        def swait(c, b):
            pltpu.make_async_copy(
                bufs[b], g_hbm.at[pl.ds(base + c * CHUNK, CHUNK)],
                ssems[b]).wait()

        for j in range(DEPTH):
            gstart(j, j)

        def body(t, carry):
            c0 = t * 2 * DEPTH
            for j in range(DEPTH):
                gwait(c0 + j, j)
                sstart(c0 + j, j)

                @pl.when(t > 0)
                def _():
                    swait(c0 - DEPTH + j, DEPTH + j)

                gstart(c0 + DEPTH + j, DEPTH + j)
            for j in range(DEPTH):
                gwait(c0 + DEPTH + j, DEPTH + j)
                sstart(c0 + DEPTH + j, DEPTH + j)
                swait(c0 + j, j)
                gstart(c0 + 2 * DEPTH + j, j)
            return carry

        lax.fori_loop(0, PAIRS, body, 0)

        c0 = PAIRS * 2 * DEPTH
        for j in range(DEPTH):
            gwait(c0 + j, j)
            sstart(c0 + j, j)
            swait(c0 - DEPTH + j, DEPTH + j)
        for j in range(DEPTH):
            swait(c0 + j, j)

    return gather_k


def _edge_body(g_ref, s_ref, h_ref, x_ref, wd_ref, we2_ref, be2_ref,
               wc1_ref, bc1_ref, wc2_ref, wn1h_ref, wn1m_ref, bn1_ref,
               wn2_ref, bn2_ref, hnew_ref, xnew_ref):
    g = g_ref[...]
    gw = lax.bitcast_convert_type(g[:, :HH], jnp.int32)
    bn_lo = lax.bitcast_convert_type(lax.shift_left(gw, 16), jnp.float32)
    bn_hi = lax.bitcast_convert_type(gw & (-65536), jnp.float32)
    bn = jnp.concatenate([bn_lo, bn_hi], axis=1)
    xj4 = g[:, HH:HH + 4]
    xi4 = x_ref[0]
    xib = jnp.broadcast_to(xi4[:, None, :], (BN, K, 4)).reshape(EB, 4)
    cross = jnp.sum(xib * xj4, axis=1, keepdims=True)
    sb = jnp.broadcast_to(
        s_ref[0][:, None, :], (BN, K, H)).reshape(EB, H)
    pre = sb + bn + (-2.0 * cross) * wd_ref[...]
    t1 = _silu_bf(pre)
    m = _silu_bf(_dotbf(t1, we2_ref[...]) + be2_ref[...])
    c1 = _silu_bf(_dotbf(m, wc1_ref[...]) + bc1_ref[...])
    wc2t = jnp.broadcast_to(wc2_ref[...], (H, 8))
    cw8 = _dotbf(c1, wc2t)
    fused = xj4 * cw8[:, :4]
    red = jnp.sum(fused.reshape(BN, K, 4), axis=1)
    m_i = jnp.sum(m.reshape(BN, K, H), axis=1,
                  dtype=jnp.float32)
    xi = xi4[:, :3]
    xnew_ref[0] = xi + (xi * red[:, 3:4] - red[:, :3]) * (1.0 / K)
    h = h_ref[0]
    z = (_dotbf(h, wn1h_ref[...]) + _dotbf(m_i, wn1m_ref[...])
         + bn1_ref[...])
    hnew_ref[0] = _dotbf(_silu(z), wn2_ref[...]) + bn2_ref[...] + h


def _const_spec(shape):
    return pl.BlockSpec(shape, lambda i: tuple(0 for _ in shape))


_prep = pl.pallas_call(
    _prep_body,
    out_shape=(
        jax.ShapeDtypeStruct((1, N, H), jnp.float32),
        jax.ShapeDtypeStruct((N, TCOLS), jnp.float32),
    ),
)


@functools.cache
def _make_edge(off):
    return pl.pallas_call(
        _edge_body,
        grid=(NBLK2,),
        in_specs=[
            pl.BlockSpec((EB, TCOLS), lambda i: (i, 0)),
            pl.BlockSpec((1, BN, H), lambda i: (0, i + off, 0)),
            pl.BlockSpec((1, BN, H), lambda i: (0, i + off, 0)),
            pl.BlockSpec((1, BN, 4), lambda i: (0, i + off, 0)),
            _const_spec((1, H)),
            _const_spec((H, H)),
            _const_spec((1, H)),
            _const_spec((H, H)),
            _const_spec((1, H)),
            _const_spec((H, 1)),
            _const_spec((H, H)),
            _const_spec((H, H)),
            _const_spec((1, H)),
            _const_spec((H, H)),
            _const_spec((1, H)),
        ],
        out_specs=(
            pl.BlockSpec((1, BN, H), lambda i: (0, i, 0)),
            pl.BlockSpec((1, BN, 3), lambda i: (0, i, 0)),
        ),
        out_shape=(
            jax.ShapeDtypeStruct((1, N2, H), jnp.float32),
            jax.ShapeDtypeStruct((1, N2, 3), jnp.float32),
        ),
    )


def kernel(h, x, edge_idx, W_e1, b_e1, W_e2, b_e2, W_c1, b_c1, W_c2,
           W_n1, b_n1, W_n2, b_n2):
    eidx = edge_idx.reshape(E)
    wd = lax.slice(W_e1, (2 * H, 0), (2 * H + 1, H))
    be1 = b_e1.reshape(1, H)
    be2 = b_e2.reshape(1, H)
    bc1 = b_c1.reshape(1, H)
    bn1 = b_n1.reshape(1, H)
    bn2 = b_n2.reshape(1, H)

    s_arr, t_arr = _prep(h, x, W_e1, be1)
    x4 = jnp.pad(x, ((0, 0), (0, 0), (0, 1)))
    wn1h = lax.slice(W_n1, (0, 0), (H, H))
    wn1m = lax.slice(W_n1, (H, 0), (2 * H, H))
    weights = (wd, W_e2, be2, W_c1, bc1, W_c2, wn1h, wn1m, bn1, W_n2, bn2)
    h_halves, x_halves = [], []
    for c in range(NCH):
        g_c = _make_gather(c * E2)(t_arr, eidx)
        hn, xn = _make_edge(c * NBLK2)(g_c, s_arr, h, x4, *weights)
        h_halves.append(hn)
        x_halves.append(xn)
    h_new = jnp.concatenate(h_halves, axis=1)
    x_new = jnp.concatenate(x_halves, axis=1)
    return (h_new, x_new)


# speedup vs baseline: 1.0481x; 1.0481x over previous
import functools

import jax
import jax.numpy as jnp
from jax import lax
from jax.experimental import pallas as pl
from jax.experimental.pallas import tpu as pltpu
from jax.experimental.pallas import tpu_sc as plsc

N = 10000
K = 32
H = 128
HH = H // 2
TCOLS = 128
E = N * K

BN = 200
EB = BN * K
NBLK = N // BN

SPLITS = ((0, 25, 5), (25, 25, 5))

NW = 32
CHUNK = 40


def _dotbf(a, b):
    return jnp.dot(a.astype(jnp.bfloat16), b.astype(jnp.bfloat16),
                   preferred_element_type=jnp.float32)


def _silu(v):
    hv = 0.5 * v
    return hv * jnp.tanh(hv) + hv


def _silu_bf(v):
    hv = jnp.bfloat16(0.5) * v.astype(jnp.bfloat16)
    return hv * jnp.tanh(hv) + hv


def _prep_body(h_ref, x_ref, we1_ref, be1_ref, s_ref, t_ref):
    h = h_ref[0]
    s_ref[0] = (
        jnp.dot(h, we1_ref[:H], preferred_element_type=jnp.float32)
        + be1_ref[...]
    )
    bn = _dotbf(h, we1_ref[H:2 * H])
    bnb = lax.bitcast_convert_type(bn, jnp.int32)
    lo16 = lax.shift_right_logical(bnb[:, :HH] + 0x8000, 16)
    hi16 = (bnb[:, HH:] + 0x8000) & (-65536)
    t_ref[:, :HH] = lax.bitcast_convert_type(hi16 | lo16, jnp.float32)
    t_ref[:, HH:HH + 3] = x_ref[0]
    t_ref[:, HH + 3:] = jnp.zeros((N, TCOLS - HH - 3), jnp.float32)


@functools.cache
def _make_gather(off, rows, depth):
    per_tile = rows // NW
    nchunks = per_tile // CHUNK
    pairs = (nchunks - depth) // (2 * depth)
    assert depth * (2 * pairs + 1) == nchunks

    mesh = plsc.VectorSubcoreMesh(core_axis_name="c", subcore_axis_name="s")
    scratch = [pltpu.VMEM((per_tile,), jnp.int32)]
    scratch += [pltpu.VMEM((CHUNK, TCOLS), jnp.float32)
                for _ in range(2 * depth)]
    scratch += [pltpu.SemaphoreType.DMA for _ in range(4 * depth)]

    @functools.partial(
        pl.kernel,
        mesh=mesh,
        out_type=jax.ShapeDtypeStruct((rows, TCOLS), jnp.float32),
        scratch_types=scratch,
    )
    def gather_k(t_hbm, eidx_hbm, g_hbm, idx_v, *rest):
        bufs = rest[:2 * depth]
        gsems = rest[2 * depth:4 * depth]
        ssems = rest[4 * depth:]
        wid = lax.axis_index("s") * 2 + lax.axis_index("c")
        base = wid * per_tile
        pltpu.sync_copy(eidx_hbm.at[pl.ds(off + base, per_tile)], idx_v)

        def gstart(c, b):
            pltpu.async_copy(
                t_hbm.at[idx_v.at[pl.ds(c * CHUNK, CHUNK)]], bufs[b],
                gsems[b])

        def gwait(c, b):
            pltpu.make_async_copy(
                t_hbm.at[idx_v.at[pl.ds(c * CHUNK, CHUNK)]], bufs[b],
                gsems[b]).wait()

        def sstart(c, b):
            pltpu.async_copy(
                bufs[b], g_hbm.at[pl.ds(base + c * CHUNK, CHUNK)], ssems[b])

        def swait(c, b):
            pltpu.make_async_copy(
                bufs[b], g_hbm.at[pl.ds(base + c * CHUNK, CHUNK)],
                ssems[b]).wait()

        for j in range(depth):
            gstart(j, j)

        def body(t, carry):
            c0 = t * 2 * depth
            for j in range(depth):
                gwait(c0 + j, j)
                sstart(c0 + j, j)

                @pl.when(t > 0)
                def _():
                    swait(c0 - depth + j, depth + j)

                gstart(c0 + depth + j, depth + j)
            for j in range(depth):
                gwait(c0 + depth + j, depth + j)
                sstart(c0 + depth + j, depth + j)
                swait(c0 + j, j)
                gstart(c0 + 2 * depth + j, j)
            return carry

        if pairs > 0:
            lax.fori_loop(0, pairs, body, 0)

        c0 = pairs * 2 * depth
        for j in range(depth):
            gwait(c0 + j, j)
            sstart(c0 + j, j)
            if pairs > 0:
                swait(c0 - depth + j, depth + j)
        for j in range(depth):
            swait(c0 + j, j)

    return gather_k


def _edge_body(g_ref, s_ref, h_ref, x_ref, wd_ref, we2_ref, be2_ref,
               wc1_ref, bc1_ref, wc2_ref, wn1h_ref, wn1m_ref, bn1_ref,
               wn2_ref, bn2_ref, hnew_ref, xnew_ref):
    g = g_ref[...]
    gw = lax.bitcast_convert_type(g[:, :HH], jnp.int32)
    bn_lo = lax.bitcast_convert_type(lax.shift_left(gw, 16), jnp.float32)
    bn_hi = lax.bitcast_convert_type(gw & (-65536), jnp.float32)
    bn = jnp.concatenate([bn_lo, bn_hi], axis=1)
    xj = g[:, HH:HH + 3]
    xi = x_ref[0]
    xib = jnp.broadcast_to(xi[:, None, :], (BN, K, 3)).reshape(EB, 3)
    diff = xib - xj
    sq = jnp.sum(diff * diff, axis=1, keepdims=True)
    sb = jnp.broadcast_to(
        s_ref[0][:, None, :], (BN, K, H)).reshape(EB, H)
    pre = sb + bn + sq * wd_ref[...]
    t1 = _silu_bf(pre)
    m = _silu_bf(_dotbf(t1, we2_ref[...]) + be2_ref[...])
    c1 = _silu_bf(_dotbf(m, wc1_ref[...]) + bc1_ref[...])
    cw = _dotbf(c1, wc2_ref[...])[:, :1]
    m_i = jnp.sum(m.reshape(BN, K, H), axis=1,
                  dtype=jnp.float32)
    xupd = jnp.sum((diff * cw).reshape(BN, K, 3), axis=1) * (1.0 / K)
    xnew_ref[0] = xi + xupd
    h = h_ref[0]
    z = (_dotbf(h, wn1h_ref[...]) + _dotbf(m_i, wn1m_ref[...])
         + bn1_ref[...])
    hnew_ref[0] = _dotbf(_silu(z), wn2_ref[...]) + bn2_ref[...] + h


def _const_spec(shape):
    return pl.BlockSpec(shape, lambda i: tuple(0 for _ in shape))


_prep = pl.pallas_call(
    _prep_body,
    out_shape=(
        jax.ShapeDtypeStruct((1, N, H), jnp.float32),
        jax.ShapeDtypeStruct((N, TCOLS), jnp.float32),
    ),
)


@functools.cache
def _make_edge(off, nblk):
    return pl.pallas_call(
        _edge_body,
        grid=(nblk,),
        in_specs=[
            pl.BlockSpec((EB, TCOLS), lambda i: (i, 0)),
            pl.BlockSpec((1, BN, H), lambda i: (0, i + off, 0)),
            pl.BlockSpec((1, BN, H), lambda i: (0, i + off, 0)),
            pl.BlockSpec((1, BN, 3), lambda i: (0, i + off, 0)),
            _const_spec((1, H)),
            _const_spec((H, H)),
            _const_spec((1, H)),
            _const_spec((H, H)),
            _const_spec((1, H)),
            _const_spec((H, 8)),
            _const_spec((H, H)),
            _const_spec((H, H)),
            _const_spec((1, H)),
            _const_spec((H, H)),
            _const_spec((1, H)),
        ],
        out_specs=(
            pl.BlockSpec((1, BN, H), lambda i: (0, i, 0)),
            pl.BlockSpec((1, BN, 3), lambda i: (0, i, 0)),
        ),
        out_shape=(
            jax.ShapeDtypeStruct((1, nblk * BN, H), jnp.float32),
            jax.ShapeDtypeStruct((1, nblk * BN, 3), jnp.float32),
        ),
    )


def kernel(h, x, edge_idx, W_e1, b_e1, W_e2, b_e2, W_c1, b_c1, W_c2,
           W_n1, b_n1, W_n2, b_n2):
    eidx = edge_idx.reshape(E)
    wd = lax.slice(W_e1, (2 * H, 0), (2 * H + 1, H))
    be1 = b_e1.reshape(1, H)
    be2 = b_e2.reshape(1, H)
    bc1 = b_c1.reshape(1, H)
    bn1 = b_n1.reshape(1, H)
    bn2 = b_n2.reshape(1, H)
    wc2p = jnp.pad(W_c2, ((0, 0), (0, 7)))
    wn1h = lax.slice(W_n1, (0, 0), (H, H))
    wn1m = lax.slice(W_n1, (H, 0), (2 * H, H))

    s_arr, t_arr = _prep(h, x, W_e1, be1)
    weights = (wd, W_e2, be2, W_c1, bc1, wc2p, wn1h, wn1m, bn1, W_n2, bn2)
    h_parts, x_parts = [], []
    for off, nblk, depth in SPLITS:
        rows = nblk * BN * K
        g_c = _make_gather(off * BN * K, rows, depth)(t_arr, eidx)
        hn, xn = _make_edge(off, nblk)(g_c, s_arr, h, x, *weights)
        h_parts.append(hn)
        x_parts.append(xn)
    h_new = jnp.concatenate(h_parts, axis=1)
    x_new = jnp.concatenate(x_parts, axis=1)
    return (h_new, x_new)


# speedup vs baseline: 1.1050x; 1.0544x over previous
import functools

import jax
import jax.numpy as jnp
from jax import lax
from jax.experimental import pallas as pl
from jax.experimental.pallas import tpu as pltpu
from jax.experimental.pallas import tpu_sc as plsc

N = 10000
K = 32
H = 128
HH = H // 2
TCOLS = 128
E = N * K

BN = 200
EB = BN * K
NBLK = N // BN

SPLITS = ((0, 10, 2), (10, 15, 5), (25, 25, 5))

NW = 32
CHUNK = 40


def _dotbf(a, b):
    return jnp.dot(a.astype(jnp.bfloat16), b.astype(jnp.bfloat16),
                   preferred_element_type=jnp.float32)


def _silu(v):
    hv = 0.5 * v
    return hv * jnp.tanh(hv) + hv


def _silu_bf(v):
    hv = jnp.bfloat16(0.5) * v.astype(jnp.bfloat16)
    return hv * jnp.tanh(hv) + hv


def _prep_body(h_ref, x_ref, we1_ref, be1_ref, s_ref, t_ref):
    h = h_ref[0]
    s_ref[0] = (
        jnp.dot(h, we1_ref[:H], preferred_element_type=jnp.float32)
        + be1_ref[...]
    )
    bn = _dotbf(h, we1_ref[H:2 * H])
    bnb = lax.bitcast_convert_type(bn, jnp.int32)
    lo16 = lax.shift_right_logical(bnb[:, :HH] + 0x8000, 16)
    hi16 = (bnb[:, HH:] + 0x8000) & (-65536)
    t_ref[:, :HH] = lax.bitcast_convert_type(hi16 | lo16, jnp.float32)
    t_ref[:, HH:HH + 3] = x_ref[0]
    t_ref[:, HH + 3:] = jnp.zeros((N, TCOLS - HH - 3), jnp.float32)


@functools.cache
def _make_gather(off, rows, depth):
    per_tile = rows // NW
    nchunks = per_tile // CHUNK
    pairs = (nchunks - depth) // (2 * depth)
    assert depth * (2 * pairs + 1) == nchunks

    mesh = plsc.VectorSubcoreMesh(core_axis_name="c", subcore_axis_name="s")
    scratch = [pltpu.VMEM((per_tile,), jnp.int32)]
    scratch += [pltpu.VMEM((CHUNK, TCOLS), jnp.float32)
                for _ in range(2 * depth)]
    scratch += [pltpu.SemaphoreType.DMA for _ in range(4 * depth)]

    @functools.partial(
        pl.kernel,
        mesh=mesh,
        out_type=jax.ShapeDtypeStruct((rows, TCOLS), jnp.float32),
        scratch_types=scratch,
    )
    def gather_k(t_hbm, eidx_hbm, g_hbm, idx_v, *rest):
        bufs = rest[:2 * depth]
        gsems = rest[2 * depth:4 * depth]
        ssems = rest[4 * depth:]
        wid = lax.axis_index("s") * 2 + lax.axis_index("c")
        base = wid * per_tile
        pltpu.sync_copy(eidx_hbm.at[pl.ds(off + base, per_tile)], idx_v)

        def gstart(c, b):
            pltpu.async_copy(
                t_hbm.at[idx_v.at[pl.ds(c * CHUNK, CHUNK)]], bufs[b],
                gsems[b])

        def gwait(c, b):
            pltpu.make_async_copy(
                t_hbm.at[idx_v.at[pl.ds(c * CHUNK, CHUNK)]], bufs[b],
                gsems[b]).wait()

        def sstart(c, b):
            pltpu.async_copy(
                bufs[b], g_hbm.at[pl.ds(base + c * CHUNK, CHUNK)], ssems[b])

        def swait(c, b):
            pltpu.make_async_copy(
                bufs[b], g_hbm.at[pl.ds(base + c * CHUNK, CHUNK)],
                ssems[b]).wait()

        for j in range(depth):
            gstart(j, j)

        def body(t, carry):
            c0 = t * 2 * depth
            for j in range(depth):
                gwait(c0 + j, j)
                sstart(c0 + j, j)

                @pl.when(t > 0)
                def _():
                    swait(c0 - depth + j, depth + j)

                gstart(c0 + depth + j, depth + j)
            for j in range(depth):
                gwait(c0 + depth + j, depth + j)
                sstart(c0 + depth + j, depth + j)
                swait(c0 + j, j)
                gstart(c0 + 2 * depth + j, j)
            return carry

        if pairs > 0:
            lax.fori_loop(0, pairs, body, 0)

        c0 = pairs * 2 * depth
        for j in range(depth):
            gwait(c0 + j, j)
            sstart(c0 + j, j)
            if pairs > 0:
                swait(c0 - depth + j, depth + j)
        for j in range(depth):
            swait(c0 + j, j)

    return gather_k


def _edge_body(g_ref, s_ref, h_ref, x_ref, wd_ref, we2_ref, be2_ref,
               wc1_ref, bc1_ref, wc2_ref, wn1h_ref, wn1m_ref, bn1_ref,
               wn2_ref, bn2_ref, hnew_ref, xnew_ref):
    g = g_ref[...]
    gw = lax.bitcast_convert_type(g[:, :HH], jnp.int32)
    bn_lo = lax.bitcast_convert_type(lax.shift_left(gw, 16), jnp.float32)
    bn_hi = lax.bitcast_convert_type(gw & (-65536), jnp.float32)
    bn = jnp.concatenate([bn_lo, bn_hi], axis=1)
    xj = g[:, HH:HH + 3]
    xi = x_ref[0]
    xib = jnp.broadcast_to(xi[:, None, :], (BN, K, 3)).reshape(EB, 3)
    diff = xib - xj
    sq = jnp.sum(diff * diff, axis=1, keepdims=True)
    sb = jnp.broadcast_to(
        s_ref[0][:, None, :], (BN, K, H)).reshape(EB, H)
    pre = sb + bn + sq * wd_ref[...]
    t1 = _silu_bf(pre)
    m = _silu_bf(_dotbf(t1, we2_ref[...]) + be2_ref[...])
    c1 = _silu_bf(_dotbf(m, wc1_ref[...]) + bc1_ref[...])
    cw = _dotbf(c1, wc2_ref[...])[:, :1]
    m_i = jnp.sum(m.reshape(BN, K, H), axis=1,
                  dtype=jnp.float32)
    xupd = jnp.sum((diff * cw).reshape(BN, K, 3), axis=1) * (1.0 / K)
    xnew_ref[0] = xi + xupd
    h = h_ref[0]
    z = (_dotbf(h, wn1h_ref[...]) + _dotbf(m_i, wn1m_ref[...])
         + bn1_ref[...])
    hnew_ref[0] = _dotbf(_silu(z), wn2_ref[...]) + bn2_ref[...] + h


def _const_spec(shape):
    return pl.BlockSpec(shape, lambda i: tuple(0 for _ in shape))


_prep = pl.pallas_call(
    _prep_body,
    out_shape=(
        jax.ShapeDtypeStruct((1, N, H), jnp.float32),
        jax.ShapeDtypeStruct((N, TCOLS), jnp.float32),
    ),
)


@functools.cache
def _make_edge(off, nblk):
    return pl.pallas_call(
        _edge_body,
        grid=(nblk,),
        in_specs=[
            pl.BlockSpec((EB, TCOLS), lambda i: (i, 0)),
            pl.BlockSpec((1, BN, H), lambda i: (0, i + off, 0)),
            pl.BlockSpec((1, BN, H), lambda i: (0, i + off, 0)),
            pl.BlockSpec((1, BN, 3), lambda i: (0, i + off, 0)),
            _const_spec((1, H)),
            _const_spec((H, H)),
            _const_spec((1, H)),
            _const_spec((H, H)),
            _const_spec((1, H)),
            _const_spec((H, 8)),
            _const_spec((H, H)),
            _const_spec((H, H)),
            _const_spec((1, H)),
            _const_spec((H, H)),
            _const_spec((1, H)),
        ],
        out_specs=(
            pl.BlockSpec((1, BN, H), lambda i: (0, i, 0)),
            pl.BlockSpec((1, BN, 3), lambda i: (0, i, 0)),
        ),
        out_shape=(
            jax.ShapeDtypeStruct((1, nblk * BN, H), jnp.float32),
            jax.ShapeDtypeStruct((1, nblk * BN, 3), jnp.float32),
        ),
    )


def kernel(h, x, edge_idx, W_e1, b_e1, W_e2, b_e2, W_c1, b_c1, W_c2,
           W_n1, b_n1, W_n2, b_n2):
    eidx = edge_idx.reshape(E)
    wd = lax.slice(W_e1, (2 * H, 0), (2 * H + 1, H))
    be1 = b_e1.reshape(1, H)
    be2 = b_e2.reshape(1, H)
    bc1 = b_c1.reshape(1, H)
    bn1 = b_n1.reshape(1, H)
    bn2 = b_n2.reshape(1, H)
    wc2p = jnp.pad(W_c2, ((0, 0), (0, 7)))
    wn1h = lax.slice(W_n1, (0, 0), (H, H))
    wn1m = lax.slice(W_n1, (H, 0), (2 * H, H))

    s_arr, t_arr = _prep(h, x, W_e1, be1)
    weights = (wd, W_e2, be2, W_c1, bc1, wc2p, wn1h, wn1m, bn1, W_n2, bn2)
    h_parts, x_parts = [], []
    for off, nblk, depth in SPLITS:
        rows = nblk * BN * K
        g_c = _make_gather(off * BN * K, rows, depth)(t_arr, eidx)
        hn, xn = _make_edge(off, nblk)(g_c, s_arr, h, x, *weights)
        h_parts.append(hn)
        x_parts.append(xn)
    h_new = jnp.concatenate(h_parts, axis=1)
    x_new = jnp.concatenate(x_parts, axis=1)
    return (h_new, x_new)
